# baseline (speedup 1.0000x reference)
import functools

import jax
import jax.numpy as jnp
import numpy as np
from jax.experimental import pallas as pl

B, S, H, D = 4, 8192, 64, 6
N = B * S

_MARKET_VEC = (0.5, -1.0, 0.8, 0.6, 1.5, 0.4)
_RISK_VEC = (0.5, -0.8, 0.6, 1.0, 1.2, 0.5)

def _noise_np():
    def rotl(x, r):
        return ((x << np.uint32(r)) | (x >> np.uint32(32 - r))).astype(np.uint32)

    ks = [np.uint32(0), np.uint32(42),
          np.uint32(0) ^ np.uint32(42) ^ np.uint32(0x1BD11BDA)]
    rotations = [[13, 15, 26, 6], [17, 29, 16, 24]]
    x0 = np.full(N, ks[0], np.uint32)
    x1 = (np.arange(N, dtype=np.uint32) + ks[1]).astype(np.uint32)
    for i in range(5):
        for r in rotations[i % 2]:
            x0 = (x0 + x1).astype(np.uint32)
            x1 = rotl(x1, r) ^ x0
        x0 = (x0 + ks[(i + 1) % 3]).astype(np.uint32)
        x1 = (x1 + ks[(i + 2) % 3] + np.uint32(i + 1)).astype(np.uint32)
    bits = x0 ^ x1

    floats = ((bits >> np.uint32(9)) | np.uint32(0x3F800000)).view(np.float32)
    floats = floats - np.float32(1.0)
    lo = np.nextafter(np.float32(-1), np.float32(0))
    u = np.maximum(lo, (floats * (np.float32(1.0) - lo) + lo).astype(np.float32))

    x = u.astype(np.float64)
    w = -np.log((1.0 - x) * (1.0 + x))
    ws = w - 2.5
    p_small = 2.81022636e-08
    for c in (3.43273939e-07, -3.5233877e-06, -4.39150654e-06, 0.00021858087,
              -0.00125372503, -0.00417768164, 0.246640727, 1.50140941):
        p_small = p_small * ws + c
    wb = np.sqrt(np.maximum(w, 5.0)) - 3.0
    p_big = -0.000200214257
    for c in (0.000100950558, 0.00134934322, -0.00367342844, 0.00573950773,
              -0.0076224613, 0.00943887047, 1.00167406, 2.83297682):
        p_big = p_big * wb + c
    erfinv = np.where(w < 5.0, p_small, p_big) * x
    return (np.sqrt(2.0) * erfinv * 0.05).astype(np.float32).reshape(B, S)


_NOISE = _noise_np()

_SEL = np.zeros((D, H), np.float32)
_SEL[0, :] = 1.0 / H
_SEL[1, :4] = 1.0 / 4.0
_SEL[2, :6] = 1.0 / 6.0
_SEL[3, :8] = 1.0 / 8.0
_SEL[4, 6:10] = 1.0 / 4.0


def _fast_sin(x):
    k = jnp.floor(x * 0.3183098861837907 + 0.5)
    r = x - k * 3.140625 - k * 9.676535897932e-4
    ki = k.astype(jnp.int32)
    sign = jnp.where((ki & 1) == 0, 1.0, -1.0)
    r2 = r * r
    s = r + r * r2 * (-0.16666667 + r2 * (8.3333310e-3
                      + r2 * (-1.98408e-4 + r2 * 2.7526e-6)))
    return sign * s


def _fused_kernel(x_ref, w_ref, sel_ref, b_ref, mv_ref, risk_ref, noise_ref,
                  preds_ref, assign_ref, probs_ref):
    x = x_ref[...]
    TB = x.shape[2]
    wext = jnp.concatenate([w_ref[...], sel_ref[...]], axis=0)
    stats = jax.lax.dot_general(
        wext, x, (((1,), (1,)), ((), ())),
        preferred_element_type=jnp.float32)

    def _dvec(vals):
        di = jax.lax.broadcasted_iota(jnp.int32, (D, 1, 1), 0)
        out = jnp.full((D, 1, 1), vals[0] * 0.3, jnp.float32)
        for k in range(1, D):
            out = jnp.where(di == k, vals[k] * 0.3, out)
        return out

    mv = mv_ref[...].reshape(1, B, TB)
    risk = risk_ref[...]
    logits = (stats[0:D] + b_ref[...][:, :, None]
              + mv * _dvec(_MARKET_VEC)
              + risk * _dvec(_RISK_VEC))
    m = stats[D:D + 1]
    m4 = stats[D + 1:D + 2]
    m6 = stats[D + 2:D + 3]
    m8 = stats[D + 3:D + 4]
    m610 = stats[D + 4:D + 5]

    sumsq = jnp.sum(x * x, axis=1, keepdims=True).reshape(1, B, TB)
    var = (sumsq * (1.0 / H) - m * m) * (H / (H - 1.0))
    std = jnp.sqrt(jnp.maximum(var, 0.0))

    lmax = jnp.max(logits, axis=0, keepdims=True)
    z = jnp.exp((logits - lmax) * (1.0 / 0.6))
    probs_ref[...] = z / jnp.sum(z, axis=0, keepdims=True)

    iota = jax.lax.broadcasted_iota(jnp.int32, logits.shape, 0)
    idx = jnp.min(jnp.where(logits == lmax, iota, D), axis=0, keepdims=True)
    assign_ref[...] = idx.reshape(B, TB)

    t3 = jnp.tanh(jnp.concatenate([m4, m8, std], axis=0))
    sig = jax.nn.sigmoid(m)
    d0 = t3[0:1] * (1.0 + std)
    d1 = sig * 0.3 - 0.15
    d2 = m6 * 0.8 + _fast_sin(m610 * 3.14159) * 0.4
    d3 = t3[1:2] * 0.9 + noise_ref[...].reshape(1, B, TB)
    d4 = jnp.maximum(m, 0.0) ** 1.2 + std * 2.5 - 0.5
    d5 = sig * 0.4 + t3[2:3] * 0.2

    preds = jnp.where(idx == 0, d0, 0.0)
    preds = jnp.where(idx == 1, d1, preds)
    preds = jnp.where(idx == 2, d2, preds)
    preds = jnp.where(idx == 3, d3, preds)
    preds = jnp.where(idx == 4, d4, preds)
    preds_ref[...] = jnp.where(idx == 5, d5, preds).reshape(B, 1, TB)


@functools.partial(jax.jit, static_argnames=("interpret",))
def kernel(sequence_embeddings, market_volatility, risk_factors, W, b,
           interpret=False):
    xt = sequence_embeddings.transpose(0, 2, 1)
    riskt = risk_factors.transpose(2, 0, 1)

    TB = 1024
    grid = (S // TB,)
    preds, assign, probs = pl.pallas_call(
        _fused_kernel,
        grid=grid,
        in_specs=[
            pl.BlockSpec((B, H, TB), lambda j: (0, 0, j)),
            pl.BlockSpec((D, H), lambda j: (0, 0)),
            pl.BlockSpec((D, H), lambda j: (0, 0)),
            pl.BlockSpec((D, 1), lambda j: (0, 0)),
            pl.BlockSpec((B, TB), lambda j: (0, j)),
            pl.BlockSpec((D, B, TB), lambda j: (0, 0, j)),
            pl.BlockSpec((B, TB), lambda j: (0, j)),
        ],
        out_specs=[
            pl.BlockSpec((B, 1, TB), lambda j: (0, 0, j)),
            pl.BlockSpec((B, TB), lambda j: (0, j)),
            pl.BlockSpec((D, B, TB), lambda j: (0, 0, j)),
        ],
        out_shape=[
            jax.ShapeDtypeStruct((B, 1, S), jnp.float32),
            jax.ShapeDtypeStruct((B, S), jnp.int32),
            jax.ShapeDtypeStruct((D, B, S), jnp.float32),
        ],
        interpret=interpret,
    )(xt, W, jnp.asarray(_SEL), b.reshape(D, 1), market_volatility,
      riskt, jnp.asarray(_NOISE.reshape(B, S)))

    return (preds.transpose(0, 2, 1),
            assign,
            probs.transpose(1, 2, 0))


# speedup vs baseline: 1.0376x; 1.0376x over previous
import functools

import jax
import jax.numpy as jnp
import numpy as np
from jax.experimental import pallas as pl

B, S, H, D = 4, 8192, 64, 6
N = B * S

_MARKET_VEC = (0.5, -1.0, 0.8, 0.6, 1.5, 0.4)
_RISK_VEC = (0.5, -0.8, 0.6, 1.0, 1.2, 0.5)

def _noise_np():
    def rotl(x, r):
        return ((x << np.uint32(r)) | (x >> np.uint32(32 - r))).astype(np.uint32)

    ks = [np.uint32(0), np.uint32(42),
          np.uint32(0) ^ np.uint32(42) ^ np.uint32(0x1BD11BDA)]
    rotations = [[13, 15, 26, 6], [17, 29, 16, 24]]
    x0 = np.full(N, ks[0], np.uint32)
    x1 = (np.arange(N, dtype=np.uint32) + ks[1]).astype(np.uint32)
    for i in range(5):
        for r in rotations[i % 2]:
            x0 = (x0 + x1).astype(np.uint32)
            x1 = rotl(x1, r) ^ x0
        x0 = (x0 + ks[(i + 1) % 3]).astype(np.uint32)
        x1 = (x1 + ks[(i + 2) % 3] + np.uint32(i + 1)).astype(np.uint32)
    bits = x0 ^ x1

    floats = ((bits >> np.uint32(9)) | np.uint32(0x3F800000)).view(np.float32)
    floats = floats - np.float32(1.0)
    lo = np.nextafter(np.float32(-1), np.float32(0))
    u = np.maximum(lo, (floats * (np.float32(1.0) - lo) + lo).astype(np.float32))

    x = u.astype(np.float64)
    w = -np.log((1.0 - x) * (1.0 + x))
    ws = w - 2.5
    p_small = 2.81022636e-08
    for c in (3.43273939e-07, -3.5233877e-06, -4.39150654e-06, 0.00021858087,
              -0.00125372503, -0.00417768164, 0.246640727, 1.50140941):
        p_small = p_small * ws + c
    wb = np.sqrt(np.maximum(w, 5.0)) - 3.0
    p_big = -0.000200214257
    for c in (0.000100950558, 0.00134934322, -0.00367342844, 0.00573950773,
              -0.0076224613, 0.00943887047, 1.00167406, 2.83297682):
        p_big = p_big * wb + c
    erfinv = np.where(w < 5.0, p_small, p_big) * x
    return (np.sqrt(2.0) * erfinv * 0.05).astype(np.float32).reshape(B, S)


_NOISE = _noise_np()

_SEL = np.zeros((D, H), np.float32)
_SEL[0, :] = 1.0 / H
_SEL[1, :4] = 1.0 / 4.0
_SEL[2, :6] = 1.0 / 6.0
_SEL[3, :8] = 1.0 / 8.0
_SEL[4, 6:10] = 1.0 / 4.0


def _fast_sin(x):
    k = jnp.floor(x * 0.3183098861837907 + 0.5)
    r = x - k * 3.140625 - k * 9.676535897932e-4
    ki = k.astype(jnp.int32)
    sign = jnp.where((ki & 1) == 0, 1.0, -1.0)
    r2 = r * r
    s = r + r * r2 * (-0.16666667 + r2 * (8.3333310e-3
                      + r2 * (-1.98408e-4 + r2 * 2.7526e-6)))
    return sign * s


def _fused_kernel(x_ref, w_ref, sel_ref, b_ref, mv_ref, risk_ref, noise_ref,
                  preds_ref, assign_ref, probs_ref):
    x = x_ref[...]
    TB = x.shape[2]
    wext = jnp.concatenate([w_ref[...], sel_ref[...]], axis=0)
    stats = jax.lax.dot_general(
        wext, x, (((1,), (1,)), ((), ())),
        preferred_element_type=jnp.float32)

    def _dvec(vals):
        di = jax.lax.broadcasted_iota(jnp.int32, (D, 1, 1), 0)
        out = jnp.full((D, 1, 1), vals[0] * 0.3, jnp.float32)
        for k in range(1, D):
            out = jnp.where(di == k, vals[k] * 0.3, out)
        return out

    mv = mv_ref[...].reshape(1, B, TB)
    risk = risk_ref[...]
    logits = (stats[0:D] + b_ref[...][:, :, None]
              + mv * _dvec(_MARKET_VEC)
              + risk * _dvec(_RISK_VEC))
    m = stats[D:D + 1]
    m4 = stats[D + 1:D + 2]
    m6 = stats[D + 2:D + 3]
    m8 = stats[D + 3:D + 4]
    m610 = stats[D + 4:D + 5]

    sumsq = jnp.sum(x * x, axis=1, keepdims=True).reshape(1, B, TB)
    var = (sumsq * (1.0 / H) - m * m) * (H / (H - 1.0))
    std = jnp.sqrt(jnp.maximum(var, 0.0))

    lmax = jnp.max(logits, axis=0, keepdims=True)
    z = jnp.exp((logits - lmax) * (1.0 / 0.6))
    probs_ref[...] = z / jnp.sum(z, axis=0, keepdims=True)

    iota = jax.lax.broadcasted_iota(jnp.int32, logits.shape, 0)
    idx = jnp.min(jnp.where(logits == lmax, iota, D), axis=0, keepdims=True)
    assign_ref[...] = idx.reshape(B, TB)

    t3 = jnp.tanh(jnp.concatenate([m4, m8, std], axis=0))
    sig = jax.nn.sigmoid(m)
    d0 = t3[0:1] * (1.0 + std)
    d1 = sig * 0.3 - 0.15
    d2 = m6 * 0.8 + _fast_sin(m610 * 3.14159) * 0.4
    d3 = t3[1:2] * 0.9 + noise_ref[...].reshape(1, B, TB)
    d4 = jnp.maximum(m, 0.0) ** 1.2 + std * 2.5 - 0.5
    d5 = sig * 0.4 + t3[2:3] * 0.2

    preds = jnp.where(idx == 0, d0, 0.0)
    preds = jnp.where(idx == 1, d1, preds)
    preds = jnp.where(idx == 2, d2, preds)
    preds = jnp.where(idx == 3, d3, preds)
    preds = jnp.where(idx == 4, d4, preds)
    preds_ref[...] = jnp.where(idx == 5, d5, preds).reshape(B, 1, TB)


@functools.partial(jax.jit, static_argnames=("interpret",))
def kernel(sequence_embeddings, market_volatility, risk_factors, W, b,
           interpret=False):
    xt = sequence_embeddings.transpose(0, 2, 1)
    riskt = risk_factors.transpose(2, 0, 1)

    TB = 4096
    grid = (S // TB,)
    preds, assign, probs = pl.pallas_call(
        _fused_kernel,
        grid=grid,
        in_specs=[
            pl.BlockSpec((B, H, TB), lambda j: (0, 0, j)),
            pl.BlockSpec((D, H), lambda j: (0, 0)),
            pl.BlockSpec((D, H), lambda j: (0, 0)),
            pl.BlockSpec((D, 1), lambda j: (0, 0)),
            pl.BlockSpec((B, TB), lambda j: (0, j)),
            pl.BlockSpec((D, B, TB), lambda j: (0, 0, j)),
            pl.BlockSpec((B, TB), lambda j: (0, j)),
        ],
        out_specs=[
            pl.BlockSpec((B, 1, TB), lambda j: (0, 0, j)),
            pl.BlockSpec((B, TB), lambda j: (0, j)),
            pl.BlockSpec((D, B, TB), lambda j: (0, 0, j)),
        ],
        out_shape=[
            jax.ShapeDtypeStruct((B, 1, S), jnp.float32),
            jax.ShapeDtypeStruct((B, S), jnp.int32),
            jax.ShapeDtypeStruct((D, B, S), jnp.float32),
        ],
        interpret=interpret,
    )(xt, W, jnp.asarray(_SEL), b.reshape(D, 1), market_volatility,
      riskt, jnp.asarray(_NOISE.reshape(B, S)))

    return (preds.transpose(0, 2, 1),
            assign,
            probs.transpose(1, 2, 0))


# speedup vs baseline: 1.1231x; 1.0824x over previous
import functools

import jax
import jax.numpy as jnp
import numpy as np
from jax.experimental import pallas as pl

B, S, H, D = 4, 8192, 64, 6
N = B * S

_MARKET_VEC = (0.5, -1.0, 0.8, 0.6, 1.5, 0.4)
_RISK_VEC = (0.5, -0.8, 0.6, 1.0, 1.2, 0.5)

def _noise_np():
    def rotl(x, r):
        return ((x << np.uint32(r)) | (x >> np.uint32(32 - r))).astype(np.uint32)

    ks = [np.uint32(0), np.uint32(42),
          np.uint32(0) ^ np.uint32(42) ^ np.uint32(0x1BD11BDA)]
    rotations = [[13, 15, 26, 6], [17, 29, 16, 24]]
    x0 = np.full(N, ks[0], np.uint32)
    x1 = (np.arange(N, dtype=np.uint32) + ks[1]).astype(np.uint32)
    for i in range(5):
        for r in rotations[i % 2]:
            x0 = (x0 + x1).astype(np.uint32)
            x1 = rotl(x1, r) ^ x0
        x0 = (x0 + ks[(i + 1) % 3]).astype(np.uint32)
        x1 = (x1 + ks[(i + 2) % 3] + np.uint32(i + 1)).astype(np.uint32)
    bits = x0 ^ x1

    floats = ((bits >> np.uint32(9)) | np.uint32(0x3F800000)).view(np.float32)
    floats = floats - np.float32(1.0)
    lo = np.nextafter(np.float32(-1), np.float32(0))
    u = np.maximum(lo, (floats * (np.float32(1.0) - lo) + lo).astype(np.float32))

    x = u.astype(np.float64)
    w = -np.log((1.0 - x) * (1.0 + x))
    ws = w - 2.5
    p_small = 2.81022636e-08
    for c in (3.43273939e-07, -3.5233877e-06, -4.39150654e-06, 0.00021858087,
              -0.00125372503, -0.00417768164, 0.246640727, 1.50140941):
        p_small = p_small * ws + c
    wb = np.sqrt(np.maximum(w, 5.0)) - 3.0
    p_big = -0.000200214257
    for c in (0.000100950558, 0.00134934322, -0.00367342844, 0.00573950773,
              -0.0076224613, 0.00943887047, 1.00167406, 2.83297682):
        p_big = p_big * wb + c
    erfinv = np.where(w < 5.0, p_small, p_big) * x
    return (np.sqrt(2.0) * erfinv * 0.05).astype(np.float32).reshape(B, S)


_NOISE = _noise_np()

_SEL = np.zeros((D, H), np.float32)
_SEL[0, :] = 1.0 / H
_SEL[1, :4] = 1.0 / 4.0
_SEL[2, :6] = 1.0 / 6.0
_SEL[3, :8] = 1.0 / 8.0
_SEL[4, 6:10] = 1.0 / 4.0


def _fast_sin(x):
    k = jnp.floor(x * 0.3183098861837907 + 0.5)
    r = x - k * 3.140625 - k * 9.676535897932e-4
    ki = k.astype(jnp.int32)
    sign = jnp.where((ki & 1) == 0, 1.0, -1.0)
    r2 = r * r
    s = r + r * r2 * (-0.16666667 + r2 * (8.3333310e-3
                      + r2 * (-1.98408e-4 + r2 * 2.7526e-6)))
    return sign * s


def _fused_kernel(x_ref, w_ref, sel_ref, b_ref, mv_ref, risk_ref, noise_ref,
                  preds_ref, assign_ref, probs_ref):
    x = x_ref[...]
    TB = x.shape[2]
    wext = jnp.concatenate([w_ref[...], sel_ref[...]], axis=0)
    stats = jax.lax.dot_general(
        wext, x, (((1,), (1,)), ((), ())),
        preferred_element_type=jnp.float32)

    def _dvec(vals):
        di = jax.lax.broadcasted_iota(jnp.int32, (D, 1, 1), 0)
        out = jnp.full((D, 1, 1), vals[0] * 0.3, jnp.float32)
        for k in range(1, D):
            out = jnp.where(di == k, vals[k] * 0.3, out)
        return out

    mv = mv_ref[...].reshape(1, B, TB)
    risk = risk_ref[...]
    logits = (stats[0:D] + b_ref[...][:, :, None]
              + mv * _dvec(_MARKET_VEC)
              + risk * _dvec(_RISK_VEC))
    m = stats[D:D + 1]
    m4 = stats[D + 1:D + 2]
    m6 = stats[D + 2:D + 3]
    m8 = stats[D + 3:D + 4]
    m610 = stats[D + 4:D + 5]

    sumsq = jnp.sum(x * x, axis=1, keepdims=True).reshape(1, B, TB)
    var = (sumsq * (1.0 / H) - m * m) * (H / (H - 1.0))
    std = jnp.sqrt(jnp.maximum(var, 0.0))

    lmax = jnp.max(logits, axis=0, keepdims=True)
    z = jnp.exp((logits - lmax) * (1.0 / 0.6))
    probs_ref[...] = z / jnp.sum(z, axis=0, keepdims=True)

    iota = jax.lax.broadcasted_iota(jnp.int32, logits.shape, 0)
    idx = jnp.min(jnp.where(logits == lmax, iota, D), axis=0, keepdims=True)
    assign_ref[...] = idx.reshape(B, TB)

    t3 = jnp.tanh(jnp.concatenate([m4, m8, std], axis=0))
    sig = jax.nn.sigmoid(m)
    d0 = t3[0:1] * (1.0 + std)
    d1 = sig * 0.3 - 0.15
    d2 = m6 * 0.8 + _fast_sin(m610 * 3.14159) * 0.4
    d3 = t3[1:2] * 0.9 + noise_ref[...].reshape(1, B, TB)
    d4 = jnp.maximum(m, 0.0) ** 1.2 + std * 2.5 - 0.5
    d5 = sig * 0.4 + t3[2:3] * 0.2

    preds = jnp.where(idx == 0, d0, 0.0)
    preds = jnp.where(idx == 1, d1, preds)
    preds = jnp.where(idx == 2, d2, preds)
    preds = jnp.where(idx == 3, d3, preds)
    preds = jnp.where(idx == 4, d4, preds)
    preds_ref[...] = jnp.where(idx == 5, d5, preds).reshape(B, 1, TB)


@functools.partial(jax.jit, static_argnames=("interpret",))
def kernel(sequence_embeddings, market_volatility, risk_factors, W, b,
           interpret=False):
    xt = sequence_embeddings.transpose(0, 2, 1)
    riskt = risk_factors.transpose(2, 0, 1)

    TB = 2048
    grid = (S // TB,)
    preds, assign, probs = pl.pallas_call(
        _fused_kernel,
        grid=grid,
        in_specs=[
            pl.BlockSpec((B, H, TB), lambda j: (0, 0, j)),
            pl.BlockSpec((D, H), lambda j: (0, 0)),
            pl.BlockSpec((D, H), lambda j: (0, 0)),
            pl.BlockSpec((D, 1), lambda j: (0, 0)),
            pl.BlockSpec((B, TB), lambda j: (0, j)),
            pl.BlockSpec((D, B, TB), lambda j: (0, 0, j)),
            pl.BlockSpec((B, TB), lambda j: (0, j)),
        ],
        out_specs=[
            pl.BlockSpec((B, 1, TB), lambda j: (0, 0, j)),
            pl.BlockSpec((B, TB), lambda j: (0, j)),
            pl.BlockSpec((D, B, TB), lambda j: (0, 0, j)),
        ],
        out_shape=[
            jax.ShapeDtypeStruct((B, 1, S), jnp.float32),
            jax.ShapeDtypeStruct((B, S), jnp.int32),
            jax.ShapeDtypeStruct((D, B, S), jnp.float32),
        ],
        interpret=interpret,
    )(xt, W, jnp.asarray(_SEL), b.reshape(D, 1), market_volatility,
      riskt, jnp.asarray(_NOISE.reshape(B, S)))

    return (preds.transpose(0, 2, 1),
            assign,
            probs.transpose(1, 2, 0))
